# Initial kernel scaffold; baseline (speedup 1.0000x reference)
import functools

import numpy as np

import jax
import jax.numpy as jnp
from jax import lax
from jax.experimental import pallas as pl
from jax.experimental.pallas import tpu as pltpu
from jax.experimental.pallas import tpu_sc as plsc

N = 100000
Q = 0.25
F = 16
MW = 128
KCH = 8
NW = 32
ZR = 6256
NACC = 16 * ZR
NP8 = N * F // 128
BLK = 1024


def _splat(vec, t):
  idx = jnp.full((16, 1), t, dtype=jnp.int32)
  return lax.gather(
      vec, idx,
      lax.GatherDimensionNumbers(
          offset_dims=(), collapsed_slice_dims=(0,), start_index_map=(0,)),
      (1,), mode=lax.GatherScatterMode.PROMISE_IN_BOUNDS)


@functools.cache
def _make_sc_pass(nmac):
  mw_per_worker = nmac * KCH
  mesh = plsc.VectorSubcoreMesh(core_axis_name="c", subcore_axis_name="s")

  @functools.partial(
      pl.kernel,
      out_type=jax.ShapeDtypeStruct((2, NACC, F), jnp.float32),
      mesh=mesh,
      scratch_types=[
          pltpu.VMEM((KCH, MW), jnp.int32),
          pltpu.VMEM((KCH, MW), jnp.int32),
          pltpu.VMEM((KCH, MW), jnp.float32),
          pltpu.VMEM((KCH, MW, F), jnp.float32),
          pltpu.VMEM((ZR // 2, F), jnp.float32),
          pltpu.VMEM_SHARED((NACC, F), jnp.float32),
          pltpu.SemaphoreType.DMA,
          pltpu.SemaphoreType.DMA,
          pltpu.SemaphoreType.DMA,
      ],
  )
  def sc_pass(src_hbm, dst_hbm, w_hbm, table_hbm, zeros_hbm, out_hbm,
              srcv, dstv, wv, rowsv, fbuf, acc, sem_i, sem_g, sem_s):
    cid = lax.axis_index("c")
    sid = lax.axis_index("s")
    wid = sid * 2 + cid

    pltpu.sync_copy(zeros_hbm, acc.at[pl.ds(sid * ZR, ZR)])
    plsc.subcore_barrier()

    base = wid * mw_per_worker

    def macro(mi, carry):
      row0 = base + mi * KCH
      ci = pltpu.async_copy(src_hbm.at[pl.ds(row0, KCH)], srcv, sem_i)
      cd = pltpu.async_copy(dst_hbm.at[pl.ds(row0, KCH)], dstv, sem_i)
      cw = pltpu.async_copy(w_hbm.at[pl.ds(row0, KCH)], wv, sem_i)
      ci.wait()
      gds = [pltpu.async_copy(table_hbm.at[srcv.at[j]], rowsv.at[j], sem_g)
             for j in range(KCH)]
      cd.wait()
      cw.wait()
      sds = []
      for j in range(KCH):
        gds[j].wait()

        def scale_group(g, c, j=j):
          wvec = wv[j, pl.ds(g * 16, 16)]
          for t in range(16):
            e = g * 16 + t
            rowsv[j, e, :] = rowsv[j, e, :] * _splat(wvec, t)
          return c

        lax.fori_loop(0, MW // 16, scale_group, 0)
        sds.append(
            pltpu.async_copy(rowsv.at[j], acc.at[dstv.at[j]], sem_s,
                             add=True))
      for d in sds:
        d.wait()
      return carry

    lax.fori_loop(0, nmac, macro, 0)
    plsc.subcore_barrier()

    half = ZR // 2
    for h in range(2):
      start = sid * ZR + h * half
      pltpu.sync_copy(acc.at[pl.ds(start, half)], fbuf)
      pltpu.sync_copy(fbuf, out_hbm.at[cid].at[pl.ds(start, half)])

  return sc_pass


def _row_spec():
  return pl.BlockSpec((BLK, 128), lambda i: (i, 0))


def _full_spec(shape):
  return pl.BlockSpec(shape, lambda i: tuple(0 for _ in shape))


def _add2(a, b):
  def body(a_ref, b_ref, o_ref):
    o_ref[...] = a_ref[...] + b_ref[...]

  return pl.pallas_call(
      body,
      out_shape=jax.ShapeDtypeStruct(a.shape, a.dtype),
      grid=(pl.cdiv(a.shape[0], BLK),),
      in_specs=[_row_spec(), _row_spec()],
      out_specs=_row_spec())(a, b)


def _layer1(x_p, a_p, d_p, e0, e1, f0, f1, w0, w1, w2, bb):
  def body(x_ref, a_ref, d_ref, e0_ref, e1_ref, f0_ref, f1_ref,
           w0_ref, w1_ref, w2_ref, b_ref, or_ref, oi_ref):
    x = x_ref[...]
    a = a_ref[...]
    d = d_ref[...]
    e = e0_ref[...] + e1_ref[...]
    f = f0_ref[...] + f1_ref[...]
    dot = lambda u, wref: jnp.dot(u, wref[...],
                                  preferred_element_type=jnp.float32)
    t0 = dot(x, w0_ref)
    outr = t0 + dot(a - d, w1_ref) + dot(2.0 * e - 2.0 * f, w2_ref) + b_ref[...]
    outi = t0 + dot(a + d, w1_ref) + dot(2.0 * e + 2.0 * f - 2.0 * x, w2_ref) \
        + b_ref[...]
    or_ref[...] = jnp.maximum(outr, 0.0)
    oi_ref[...] = outi

  out = jax.ShapeDtypeStruct((NP8, 128), jnp.float32)
  return pl.pallas_call(
      body,
      out_shape=(out, out),
      grid=(pl.cdiv(NP8, BLK),),
      in_specs=[_row_spec()] * 7 + [_full_spec((128, 128))] * 3
      + [_full_spec((1, 128))],
      out_specs=(_row_spec(), _row_spec()))(
          x_p, a_p, d_p, e0, e1, f0, f1, w0, w1, w2, bb)


def _layer2(xr_p, xi_p, a_p, b_p, c_p, d_p, e0, e1, f0, f1, g0, g1, h0, h1,
            w0, w1, w2, bb):
  def body(xr_ref, xi_ref, a_ref, b_ref, c_ref, d_ref, e0_ref, e1_ref,
           f0_ref, f1_ref, g0_ref, g1_ref, h0_ref, h1_ref,
           w0_ref, w1_ref, w2_ref, b_ref, or_ref, oi_ref):
    xr = xr_ref[...]
    xi = xi_ref[...]
    a = a_ref[...]
    b = b_ref[...]
    c = c_ref[...]
    d = d_ref[...]
    e = e0_ref[...] + e1_ref[...]
    f = f0_ref[...] + f1_ref[...]
    g = g0_ref[...] + g1_ref[...]
    h = h0_ref[...] + h1_ref[...]
    dot = lambda u, wref: jnp.dot(u, wref[...],
                                  preferred_element_type=jnp.float32)
    outr = dot(xr, w0_ref) + dot(a - d, w1_ref) \
        + dot(2.0 * e - xr - 2.0 * f + xi, w2_ref) + b_ref[...]
    outi = dot(xi, w0_ref) + dot(b + c, w1_ref) \
        + dot(2.0 * g - xi + 2.0 * h - xr, w2_ref) + b_ref[...]
    or_ref[...] = jnp.maximum(outr, 0.0)
    oi_ref[...] = outi

  out = jax.ShapeDtypeStruct((NP8, 128), jnp.float32)
  return pl.pallas_call(
      body,
      out_shape=(out, out),
      grid=(pl.cdiv(NP8, BLK),),
      in_specs=[_row_spec()] * 14 + [_full_spec((128, 128))] * 3
      + [_full_spec((1, 128))],
      out_specs=(_row_spec(), _row_spec()))(
          xr_p, xi_p, a_p, b_p, c_p, d_p, e0, e1, f0, f1, g0, g1, h0, h1,
          w0, w1, w2, bb)


def _layer3_head(xr_p, xi_p, a_p, b_p, c_p, d_p, e0, e1, f0, f1, g0, g1,
                 h0, h1, w0, w1, w2, bb, arh, aih, blb):
  def body(xr_ref, xi_ref, a_ref, b_ref, c_ref, d_ref, e0_ref, e1_ref,
           f0_ref, f1_ref, g0_ref, g1_ref, h0_ref, h1_ref,
           w0_ref, w1_ref, w2_ref, b_ref, arh_ref, aih_ref, blb_ref, o_ref):
    xr = xr_ref[...]
    xi = xi_ref[...]
    a = a_ref[...]
    b = b_ref[...]
    c = c_ref[...]
    d = d_ref[...]
    e = e0_ref[...] + e1_ref[...]
    f = f0_ref[...] + f1_ref[...]
    g = g0_ref[...] + g1_ref[...]
    h = h0_ref[...] + h1_ref[...]
    dot = lambda u, wref: jnp.dot(u, wref[...],
                                  preferred_element_type=jnp.float32)
    r3 = dot(xr, w0_ref) + dot(a - d, w1_ref) \
        + dot(2.0 * e - xr - 2.0 * f + xi, w2_ref) + b_ref[...]
    i3 = dot(xi, w0_ref) + dot(b + c, w1_ref) \
        + dot(2.0 * g - xi + 2.0 * h - xr, w2_ref) + b_ref[...]
    o_ref[...] = dot(r3, arh_ref) + dot(i3, aih_ref) + blb_ref[...]

  out = jax.ShapeDtypeStruct((NP8, 128), jnp.float32)
  return pl.pallas_call(
      body,
      out_shape=out,
      grid=(pl.cdiv(NP8, BLK),),
      in_specs=[_row_spec()] * 14 + [_full_spec((128, 128))] * 3
      + [_full_spec((1, 128))] + [_full_spec((128, 128))] * 2
      + [_full_spec((1, 128))],
      out_specs=_row_spec())(
          xr_p, xi_p, a_p, b_p, c_p, d_p, e0, e1, f0, f1, g0, g1, h0, h1,
          w0, w1, w2, bb, arh, aih, blb)


def _build_edges(edge_index):
  e_cnt = edge_index.shape[1]
  m = 2 * e_cnt
  row = edge_index[0]
  col = edge_index[1]
  r = jnp.concatenate([row, col])
  c = jnp.concatenate([col, row])
  w_dir = jnp.concatenate([jnp.ones((e_cnt,), jnp.float32),
                           -jnp.ones((e_cnt,), jnp.float32)])
  keys = r.astype(jnp.int64) * N + c.astype(jnp.int64)
  fill = jnp.int64(N) * jnp.int64(N)
  uniq, inv = jnp.unique(keys, return_inverse=True, size=m, fill_value=fill)
  inv = inv.reshape(-1)
  ws = jax.ops.segment_sum(jnp.full((m,), 0.5, jnp.float32), inv,
                           num_segments=m)
  theta = jnp.float32(2.0 * np.pi * Q) * jax.ops.segment_sum(
      w_dir, inv, num_segments=m)
  pad = uniq == fill
  ur = jnp.where(pad, N, uniq // N).astype(jnp.int32)
  uc = jnp.where(pad, N, uniq % N).astype(jnp.int32)
  deg = jax.ops.segment_sum(ws, ur, num_segments=N)
  dinv = jnp.where(deg > 0, 1.0 / jnp.sqrt(jnp.maximum(deg, 1e-30)), 0.0)
  a_norm = dinv[jnp.minimum(ur, N - 1)] * ws * dinv[jnp.minimum(uc, N - 1)]
  nr = (-a_norm * jnp.cos(theta)).astype(jnp.float32)
  ni = (-a_norm * jnp.sin(theta)).astype(jnp.float32)

  idx = jnp.arange(m, dtype=jnp.int32)
  gsrc = jnp.where(pad, idx & 1023, jnp.minimum(uc, N - 1))
  gdst = jnp.where(pad, N + (idx % 96), ur)
  wr = jnp.where(pad, 0.0, nr)
  wi = jnp.where(pad, 0.0, -ni)

  nmac = -(-m // (NW * MW * KCH))
  m_pad = NW * MW * KCH * nmac
  pad_n = m_pad - m
  pidx = jnp.arange(pad_n, dtype=jnp.int32)
  gsrc = jnp.concatenate([gsrc, pidx & 1023])
  gdst = jnp.concatenate([gdst, N + (pidx % 96)])
  zf = jnp.zeros((pad_n,), jnp.float32)
  wr = jnp.concatenate([wr, zf])
  wi = jnp.concatenate([wi, zf])
  return (gsrc.reshape(-1, MW), gdst.reshape(-1, MW),
          wr.reshape(-1, MW), wi.reshape(-1, MW), nmac)


def _blockdiag(w):
  wp = jnp.pad(w, ((0, 16 - w.shape[0]), (0, 16 - w.shape[1])))
  return jnp.kron(jnp.eye(8, dtype=jnp.float32), wp)


def kernel(data_x, data_edge_index, W1, b1, W2, b2, W3, b3, Wl, bl):
  x = data_x.astype(jnp.float32)
  src2, dst2, wr2, wi2, nmac = _build_edges(data_edge_index)
  zeros = jnp.zeros((ZR, F), jnp.float32)

  w1b = [_blockdiag(W1[k]) for k in range(3)]
  w2b = [_blockdiag(W2[k]) for k in range(3)]
  w3b = [_blockdiag(W3[k]) for k in range(3)]
  b1b = jnp.tile(jnp.pad(b1, (0, 16 - b1.shape[0])), 8)[None, :]
  b2b = jnp.tile(jnp.pad(b2, (0, 16 - b2.shape[0])), 8)[None, :]
  b3b = jnp.tile(jnp.pad(b3, (0, 16 - b3.shape[0])), 8)[None, :]
  bh_r = jnp.zeros((16, 16), jnp.float32).at[:8, 0].set(Wl[:8, 0])
  bh_i = jnp.zeros((16, 16), jnp.float32).at[:8, 0].set(Wl[8:, 0])
  arh = jnp.kron(jnp.eye(8, dtype=jnp.float32), bh_r)
  aih = jnp.kron(jnp.eye(8, dtype=jnp.float32), bh_i)
  blb = jnp.tile(jnp.concatenate([bl, jnp.zeros((15,), jnp.float32)]),
                 8)[None, :]

  sc_pass = _make_sc_pass(nmac)

  def prop_partial(table, w2_):
    return sc_pass(src2, dst2, w2_, table, zeros)

  def prop(table, w2_):
    parts = prop_partial(table, w2_)
    comb_p = _add2(parts[0, :N].reshape(NP8, 128),
                   parts[1, :N].reshape(NP8, 128))
    return comb_p.reshape(N, F), comb_p

  def unpack(parts):
    return (parts[0, :N].reshape(NP8, 128), parts[1, :N].reshape(NP8, 128))

  x_p = x.reshape(NP8, 128)
  a_t, a_p = prop(x, wr2)
  d_t, d_p = prop(x, wi2)
  e0, e1 = unpack(prop_partial(a_t, wr2))
  f0, f1 = unpack(prop_partial(d_t, wi2))
  xr_p, xi_p = _layer1(x_p, a_p, d_p, e0, e1, f0, f1,
                       w1b[0], w1b[1], w1b[2], b1b)

  head_p = None
  for wlist, bb, last in ((w2b, b2b, False), (w3b, b3b, True)):
    xr_t = xr_p.reshape(N, F)
    xi_t = xi_p.reshape(N, F)
    a_t, a_p = prop(xr_t, wr2)
    b_t, b_p = prop(xi_t, wr2)
    c_t, c_p = prop(xr_t, wi2)
    d_t, d_p = prop(xi_t, wi2)
    e0, e1 = unpack(prop_partial(a_t, wr2))
    g0, g1 = unpack(prop_partial(b_t, wr2))
    h0, h1 = unpack(prop_partial(c_t, wi2))
    f0, f1 = unpack(prop_partial(d_t, wi2))
    if last:
      head_p = _layer3_head(xr_p, xi_p, a_p, b_p, c_p, d_p,
                            e0, e1, f0, f1, g0, g1, h0, h1,
                            wlist[0], wlist[1], wlist[2], bb, arh, aih, blb)
    else:
      xr_p, xi_p = _layer2(xr_p, xi_p, a_p, b_p, c_p, d_p,
                           e0, e1, f0, f1, g0, g1, h0, h1,
                           wlist[0], wlist[1], wlist[2], bb)

  return head_p.reshape(N, F)[:, :1]


# speedup vs baseline: 1.1967x; 1.1967x over previous
import functools

import numpy as np

import jax
import jax.numpy as jnp
from jax import lax
from jax.experimental import pallas as pl
from jax.experimental.pallas import tpu as pltpu
from jax.experimental.pallas import tpu_sc as plsc

N = 100000
Q = 0.25
F = 16
MW = 128
KCH = 8
NW = 32
ZR = 6256
NACC = 16 * ZR
NP8 = N * F // 128
BLK = 1024


def _splat(vec, t):
  idx = jnp.full((16, 1), t, dtype=jnp.int32)
  return lax.gather(
      vec, idx,
      lax.GatherDimensionNumbers(
          offset_dims=(), collapsed_slice_dims=(0,), start_index_map=(0,)),
      (1,), mode=lax.GatherScatterMode.PROMISE_IN_BOUNDS)


@functools.cache
def _make_sc_pass(nmac):
  mw_per_worker = nmac * KCH
  mesh = plsc.VectorSubcoreMesh(core_axis_name="c", subcore_axis_name="s")

  @functools.partial(
      pl.kernel,
      out_type=jax.ShapeDtypeStruct((2, NACC, F), jnp.float32),
      mesh=mesh,
      compiler_params=pltpu.CompilerParams(use_tc_tiling_on_sc=False),
      scratch_types=[
          pltpu.VMEM((KCH, MW), jnp.int32),
          pltpu.VMEM((KCH, MW), jnp.int32),
          pltpu.VMEM((KCH, MW), jnp.float32),
          pltpu.VMEM((KCH, MW, F), jnp.float32),
          pltpu.VMEM_SHARED((NACC, F), jnp.float32),
          pltpu.SemaphoreType.DMA,
          pltpu.SemaphoreType.DMA,
          pltpu.SemaphoreType.DMA,
      ],
  )
  def sc_pass(src_hbm, dst_hbm, w_hbm, table_hbm, zeros_hbm, out_hbm,
              srcv, dstv, wv, rowsv, acc, sem_i, sem_g, sem_s):
    cid = lax.axis_index("c").astype(jnp.int32)
    sid = lax.axis_index("s").astype(jnp.int32)
    wid = sid * jnp.int32(2) + cid

    @pl.when(sid == 0)
    def _zero():
      pltpu.sync_copy(zeros_hbm, acc)
    plsc.subcore_barrier()

    base = (wid * jnp.int32(mw_per_worker)).astype(jnp.int32)

    def macro(mi, row0):
      del mi
      row0 = pl.multiple_of(row0, 8)
      ci = pltpu.async_copy(src_hbm.at[pl.ds(row0, KCH)], srcv, sem_i)
      cd = pltpu.async_copy(dst_hbm.at[pl.ds(row0, KCH)], dstv, sem_i)
      cw = pltpu.async_copy(w_hbm.at[pl.ds(row0, KCH)], wv, sem_i)
      ci.wait()
      cd.wait()
      cw.wait()
      gds = [pltpu.async_copy(table_hbm.at[srcv.at[jnp.int32(j)]],
                              rowsv.at[jnp.int32(j)], sem_g)
             for j in range(KCH)]
      for g_ in gds:
        g_.wait()
      for j in range(KCH):

        def scale_group(eoff, j=j):
          wvec = wv[jnp.int32(j), pl.ds(eoff, 16)]
          for t in range(16):
            e = eoff + jnp.int32(t)
            rowsv[jnp.int32(j), e, :] = rowsv[jnp.int32(j), e, :] * _splat(wvec, t)
          return eoff + jnp.int32(16)

        lax.fori_loop(jnp.int32(0), jnp.int32(MW // 16),
                      lambda g, eoff, j=j: scale_group(eoff, j=j),
                      jnp.int32(0))
      sds = [pltpu.async_copy(rowsv.at[jnp.int32(j)],
                              acc.at[dstv.at[jnp.int32(j)]], sem_s, add=True)
             for j in range(KCH)]
      for d in sds:
        d.wait()
      return row0 + jnp.int32(KCH)

    lax.fori_loop(jnp.int32(0), jnp.int32(nmac), macro, base)
    plsc.subcore_barrier()

    @pl.when(sid == 0)
    def _flush():
      pltpu.sync_copy(acc, out_hbm.at[cid])

  return sc_pass


def _row_spec():
  return pl.BlockSpec((BLK, 128), lambda i: (i, jnp.int32(0)))


def _full_spec(shape):
  return pl.BlockSpec(shape, lambda i: tuple(jnp.int32(0) for _ in shape))


def _add2(a, b):
  def body(a_ref, b_ref, o_ref):
    o_ref[...] = a_ref[...] + b_ref[...]

  return pl.pallas_call(
      body,
      out_shape=jax.ShapeDtypeStruct(a.shape, a.dtype),
      grid=(pl.cdiv(a.shape[0], BLK),),
      in_specs=[_row_spec(), _row_spec()],
      out_specs=_row_spec())(a, b)


def _layer1(x_p, a_p, d_p, e0, e1, f0, f1, w0, w1, w2, bb):
  def body(x_ref, a_ref, d_ref, e0_ref, e1_ref, f0_ref, f1_ref,
           w0_ref, w1_ref, w2_ref, b_ref, or_ref, oi_ref):
    x = x_ref[...]
    a = a_ref[...]
    d = d_ref[...]
    e = e0_ref[...] + e1_ref[...]
    f = f0_ref[...] + f1_ref[...]
    dot = lambda u, wref: jnp.dot(u, wref[...],
                                  preferred_element_type=jnp.float32)
    t0 = dot(x, w0_ref)
    outr = t0 + dot(a - d, w1_ref) + dot(2.0 * e - 2.0 * f, w2_ref) + b_ref[...]
    outi = t0 + dot(a + d, w1_ref) + dot(2.0 * e + 2.0 * f - 2.0 * x, w2_ref) \
        + b_ref[...]
    or_ref[...] = jnp.maximum(outr, 0.0)
    oi_ref[...] = outi

  out = jax.ShapeDtypeStruct((NP8, 128), jnp.float32)
  return pl.pallas_call(
      body,
      out_shape=(out, out),
      grid=(pl.cdiv(NP8, BLK),),
      in_specs=[_row_spec()] * 7 + [_full_spec((128, 128))] * 3
      + [_full_spec((1, 128))],
      out_specs=(_row_spec(), _row_spec()))(
          x_p, a_p, d_p, e0, e1, f0, f1, w0, w1, w2, bb)


def _layer2(xr_p, xi_p, a_p, b_p, c_p, d_p, e0, e1, f0, f1, g0, g1, h0, h1,
            w0, w1, w2, bb):
  def body(xr_ref, xi_ref, a_ref, b_ref, c_ref, d_ref, e0_ref, e1_ref,
           f0_ref, f1_ref, g0_ref, g1_ref, h0_ref, h1_ref,
           w0_ref, w1_ref, w2_ref, bias_ref, or_ref, oi_ref):
    xr = xr_ref[...]
    xi = xi_ref[...]
    a = a_ref[...]
    b = b_ref[...]
    c = c_ref[...]
    d = d_ref[...]
    e = e0_ref[...] + e1_ref[...]
    f = f0_ref[...] + f1_ref[...]
    g = g0_ref[...] + g1_ref[...]
    h = h0_ref[...] + h1_ref[...]
    dot = lambda u, wref: jnp.dot(u, wref[...],
                                  preferred_element_type=jnp.float32)
    outr = dot(xr, w0_ref) + dot(a - d, w1_ref) \
        + dot(2.0 * e - xr - 2.0 * f + xi, w2_ref) + bias_ref[...]
    outi = dot(xi, w0_ref) + dot(b + c, w1_ref) \
        + dot(2.0 * g - xi + 2.0 * h - xr, w2_ref) + bias_ref[...]
    or_ref[...] = jnp.maximum(outr, 0.0)
    oi_ref[...] = outi

  out = jax.ShapeDtypeStruct((NP8, 128), jnp.float32)
  return pl.pallas_call(
      body,
      out_shape=(out, out),
      grid=(pl.cdiv(NP8, BLK),),
      in_specs=[_row_spec()] * 14 + [_full_spec((128, 128))] * 3
      + [_full_spec((1, 128))],
      out_specs=(_row_spec(), _row_spec()))(
          xr_p, xi_p, a_p, b_p, c_p, d_p, e0, e1, f0, f1, g0, g1, h0, h1,
          w0, w1, w2, bb)


def _layer3_head(xr_p, xi_p, a_p, b_p, c_p, d_p, e0, e1, f0, f1, g0, g1,
                 h0, h1, w0, w1, w2, bb, arh, aih, blb):
  def body(xr_ref, xi_ref, a_ref, b_ref, c_ref, d_ref, e0_ref, e1_ref,
           f0_ref, f1_ref, g0_ref, g1_ref, h0_ref, h1_ref,
           w0_ref, w1_ref, w2_ref, bias_ref, arh_ref, aih_ref, blb_ref, o_ref):
    xr = xr_ref[...]
    xi = xi_ref[...]
    a = a_ref[...]
    b = b_ref[...]
    c = c_ref[...]
    d = d_ref[...]
    e = e0_ref[...] + e1_ref[...]
    f = f0_ref[...] + f1_ref[...]
    g = g0_ref[...] + g1_ref[...]
    h = h0_ref[...] + h1_ref[...]
    dot = lambda u, wref: jnp.dot(u, wref[...],
                                  preferred_element_type=jnp.float32)
    r3 = dot(xr, w0_ref) + dot(a - d, w1_ref) \
        + dot(2.0 * e - xr - 2.0 * f + xi, w2_ref) + bias_ref[...]
    i3 = dot(xi, w0_ref) + dot(b + c, w1_ref) \
        + dot(2.0 * g - xi + 2.0 * h - xr, w2_ref) + bias_ref[...]
    o_ref[...] = dot(r3, arh_ref) + dot(i3, aih_ref) + blb_ref[...]

  out = jax.ShapeDtypeStruct((NP8, 128), jnp.float32)
  return pl.pallas_call(
      body,
      out_shape=out,
      grid=(pl.cdiv(NP8, BLK),),
      in_specs=[_row_spec()] * 14 + [_full_spec((128, 128))] * 3
      + [_full_spec((1, 128))] + [_full_spec((128, 128))] * 2
      + [_full_spec((1, 128))],
      out_specs=_row_spec())(
          xr_p, xi_p, a_p, b_p, c_p, d_p, e0, e1, f0, f1, g0, g1, h0, h1,
          w0, w1, w2, bb, arh, aih, blb)


def _build_edges(edge_index):
  e_cnt = edge_index.shape[1]
  m = 2 * e_cnt
  row = edge_index[0]
  col = edge_index[1]
  r = jnp.concatenate([row, col])
  c = jnp.concatenate([col, row])
  w_dir = jnp.concatenate([jnp.ones((e_cnt,), jnp.float32),
                           -jnp.ones((e_cnt,), jnp.float32)])
  keys = r.astype(jnp.int64) * N + c.astype(jnp.int64)
  fill = jnp.int64(N) * jnp.int64(N)
  uniq, inv = jnp.unique(keys, return_inverse=True, size=m, fill_value=fill)
  inv = inv.reshape(-1)
  ws = jax.ops.segment_sum(jnp.full((m,), 0.5, jnp.float32), inv,
                           num_segments=m)
  theta = jnp.float32(2.0 * np.pi * Q) * jax.ops.segment_sum(
      w_dir, inv, num_segments=m)
  pad = uniq == fill
  ur = jnp.where(pad, N, uniq // N).astype(jnp.int32)
  uc = jnp.where(pad, N, uniq % N).astype(jnp.int32)
  deg = jax.ops.segment_sum(ws, ur, num_segments=N)
  dinv = jnp.where(deg > 0, 1.0 / jnp.sqrt(jnp.maximum(deg, 1e-30)), 0.0)
  a_norm = dinv[jnp.minimum(ur, N - 1)] * ws * dinv[jnp.minimum(uc, N - 1)]
  nr = (-a_norm * jnp.cos(theta)).astype(jnp.float32)
  ni = (-a_norm * jnp.sin(theta)).astype(jnp.float32)

  idx = jnp.arange(m, dtype=jnp.int32)
  gsrc = jnp.where(pad, idx & 1023, jnp.minimum(uc, N - 1))
  gdst = jnp.where(pad, N + (idx % 96), ur)
  wr = jnp.where(pad, 0.0, nr)
  wi = jnp.where(pad, 0.0, -ni)

  nmac = -(-m // (NW * MW * KCH))
  m_pad = NW * MW * KCH * nmac
  pad_n = m_pad - m
  pidx = jnp.arange(pad_n, dtype=jnp.int32)
  gsrc = jnp.concatenate([gsrc, pidx & 1023])
  gdst = jnp.concatenate([gdst, N + (pidx % 96)])
  zf = jnp.zeros((pad_n,), jnp.float32)
  wr = jnp.concatenate([wr, zf])
  wi = jnp.concatenate([wi, zf])
  return (gsrc.reshape(-1, MW), gdst.reshape(-1, MW),
          wr.reshape(-1, MW), wi.reshape(-1, MW), nmac)


def _blockdiag(w):
  wp = jnp.pad(w, ((0, 16 - w.shape[0]), (0, 16 - w.shape[1])))
  return jnp.kron(jnp.eye(8, dtype=jnp.float32), wp)


def kernel(data_x, data_edge_index, W1, b1, W2, b2, W3, b3, Wl, bl):
  x = data_x.astype(jnp.float32)
  src2, dst2, wr2, wi2, nmac = _build_edges(data_edge_index)
  zeros = jnp.zeros((NACC, F), jnp.float32)

  w1b = [_blockdiag(W1[k]) for k in range(3)]
  w2b = [_blockdiag(W2[k]) for k in range(3)]
  w3b = [_blockdiag(W3[k]) for k in range(3)]
  b1b = jnp.tile(jnp.pad(b1, (0, 16 - b1.shape[0])), 8)[None, :]
  b2b = jnp.tile(jnp.pad(b2, (0, 16 - b2.shape[0])), 8)[None, :]
  b3b = jnp.tile(jnp.pad(b3, (0, 16 - b3.shape[0])), 8)[None, :]
  bh_r = jnp.zeros((16, 16), jnp.float32).at[:8, 0].set(Wl[:8, 0])
  bh_i = jnp.zeros((16, 16), jnp.float32).at[:8, 0].set(Wl[8:, 0])
  arh = jnp.kron(jnp.eye(8, dtype=jnp.float32), bh_r)
  aih = jnp.kron(jnp.eye(8, dtype=jnp.float32), bh_i)
  blb = jnp.tile(jnp.concatenate([bl, jnp.zeros((15,), jnp.float32)]),
                 8)[None, :]

  sc_pass = _make_sc_pass(nmac)

  def prop_partial(table, w2_):
    return sc_pass(src2, dst2, w2_, table, zeros)

  def prop(table, w2_):
    parts = prop_partial(table, w2_)
    comb_p = _add2(parts[0, :N].reshape(NP8, 128),
                   parts[1, :N].reshape(NP8, 128))
    return comb_p.reshape(N, F), comb_p

  def unpack(parts):
    return (parts[0, :N].reshape(NP8, 128), parts[1, :N].reshape(NP8, 128))

  x_p = x.reshape(NP8, 128)
  a_t, a_p = prop(x, wr2)
  d_t, d_p = prop(x, wi2)
  e0, e1 = unpack(prop_partial(a_t, wr2))
  f0, f1 = unpack(prop_partial(d_t, wi2))
  xr_p, xi_p = _layer1(x_p, a_p, d_p, e0, e1, f0, f1,
                       w1b[0], w1b[1], w1b[2], b1b)

  head_p = None
  for wlist, bb, last in ((w2b, b2b, False), (w3b, b3b, True)):
    xr_t = xr_p.reshape(N, F)
    xi_t = xi_p.reshape(N, F)
    a_t, a_p = prop(xr_t, wr2)
    b_t, b_p = prop(xi_t, wr2)
    c_t, c_p = prop(xr_t, wi2)
    d_t, d_p = prop(xi_t, wi2)
    e0, e1 = unpack(prop_partial(a_t, wr2))
    g0, g1 = unpack(prop_partial(b_t, wr2))
    h0, h1 = unpack(prop_partial(c_t, wi2))
    f0, f1 = unpack(prop_partial(d_t, wi2))
    if last:
      head_p = _layer3_head(xr_p, xi_p, a_p, b_p, c_p, d_p,
                            e0, e1, f0, f1, g0, g1, h0, h1,
                            wlist[0], wlist[1], wlist[2], bb, arh, aih, blb)
    else:
      xr_p, xi_p = _layer2(xr_p, xi_p, a_p, b_p, c_p, d_p,
                           e0, e1, f0, f1, g0, g1, h0, h1,
                           wlist[0], wlist[1], wlist[2], bb)

  return head_p.reshape(N, F)[:, :1]


# speedup vs baseline: 14.0825x; 11.7674x over previous
import functools

import numpy as np

import jax
import jax.numpy as jnp
from jax import lax
from jax.experimental import pallas as pl
from jax.experimental.pallas import tpu as pltpu
from jax.experimental.pallas import tpu_sc as plsc

N = 100000
Q = 0.25
F = 16
MW = 128
KCH = 8
NW = 32
ZR = 6256
NACC = 16 * ZR
NP8 = N * F // 128
BLK = 1024
SEC = 102400
ZB = 2 * SEC
NZ = 2048
TH = ZB + NZ
SECP = SEC * F // 128
THP = TH * F // 128
SBLK = 512


@functools.cache
def _make_sc_pass(nmac):
  mw_per_worker = nmac * KCH
  mesh = plsc.VectorSubcoreMesh(core_axis_name="c", subcore_axis_name="s")

  @functools.partial(
      pl.kernel,
      out_type=jax.ShapeDtypeStruct((2, NACC, F), jnp.float32),
      mesh=mesh,
      compiler_params=pltpu.CompilerParams(use_tc_tiling_on_sc=False),
      scratch_types=[
          pltpu.VMEM((KCH, MW), jnp.int32),
          pltpu.VMEM((KCH, MW), jnp.int32),
          pltpu.VMEM((KCH, MW, F), jnp.float32),
          pltpu.VMEM_SHARED((NACC, F), jnp.float32),
          pltpu.SemaphoreType.DMA,
          pltpu.SemaphoreType.DMA,
          pltpu.SemaphoreType.DMA,
      ],
  )
  def sc_pass(src_hbm, dst_hbm, table_hbm, zeros_hbm, out_hbm,
              srcv, dstv, rowsv, acc, sem_i, sem_g, sem_s):
    cid = lax.axis_index("c").astype(jnp.int32)
    sid = lax.axis_index("s").astype(jnp.int32)
    wid = sid * jnp.int32(2) + cid

    @pl.when(sid == 0)
    def _zero():
      pltpu.sync_copy(zeros_hbm, acc)
    plsc.subcore_barrier()

    base = (wid * jnp.int32(mw_per_worker)).astype(jnp.int32)

    def macro(mi, row0):
      del mi
      row0 = pl.multiple_of(row0, 8)
      ci = pltpu.async_copy(src_hbm.at[pl.ds(row0, KCH)], srcv, sem_i)
      cd = pltpu.async_copy(dst_hbm.at[pl.ds(row0, KCH)], dstv, sem_i)
      ci.wait()
      cd.wait()
      gds = [pltpu.async_copy(table_hbm.at[srcv.at[jnp.int32(j)]],
                              rowsv.at[jnp.int32(j)], sem_g)
             for j in range(KCH)]
      for g_ in gds:
        g_.wait()
      sds = [pltpu.async_copy(rowsv.at[jnp.int32(j)],
                              acc.at[dstv.at[jnp.int32(j)]], sem_s, add=True)
             for j in range(KCH)]
      for d in sds:
        d.wait()
      return row0 + jnp.int32(KCH)

    lax.fori_loop(jnp.int32(0), jnp.int32(nmac), macro, base)
    plsc.subcore_barrier()

    @pl.when(sid == 0)
    def _flush():
      pltpu.sync_copy(acc, out_hbm.at[cid])

  return sc_pass


def _row_spec(blk=BLK):
  return pl.BlockSpec((blk, 128), lambda i: (i, jnp.int32(0)))


def _full_spec(shape):
  return pl.BlockSpec(shape, lambda i: tuple(jnp.int32(0) for _ in shape))


def _mod_spec(blk, nb):
  return pl.BlockSpec((blk, 128), lambda i: (i % nb, jnp.int32(0)))


_NB = SECP // SBLK


def _stack1(x_p, dv):
  def body(x_ref, dv_ref, o_ref):
    i = pl.program_id(0)
    u = x_ref[...] * dv_ref[...] * 0.5
    val = jnp.where(i < _NB, -u, jnp.where(i < 2 * _NB, u, 0.0))
    o_ref[...] = val

  return pl.pallas_call(
      body,
      out_shape=jax.ShapeDtypeStruct((THP, 128), jnp.float32),
      grid=(pl.cdiv(THP, SBLK),),
      in_specs=[_mod_spec(SBLK, _NB), _mod_spec(SBLK, _NB)],
      out_specs=_row_spec(SBLK))(x_p, dv)


def _stackc(p0, p1, dv):
  def body(p0_ref, p1_ref, dv_ref, o_ref, a_ref):
    i = pl.program_id(0)
    s = p0_ref[...] + p1_ref[...]
    d = dv_ref[...]
    sd = s * d
    u = sd * d * 0.5
    o_ref[...] = jnp.where(i < _NB, -u, jnp.where(i < 2 * _NB, u, 0.0))
    a_ref[...] = sd

  return pl.pallas_call(
      body,
      out_shape=(jax.ShapeDtypeStruct((THP, 128), jnp.float32),
                 jax.ShapeDtypeStruct((NP8, 128), jnp.float32)),
      grid=(pl.cdiv(THP, SBLK),),
      in_specs=[_mod_spec(SBLK, _NB)] * 3,
      out_specs=(_row_spec(SBLK), _mod_spec(SBLK, _NB)))(p0, p1, dv)


def _layer1(x_p, a_p, d_p, e0, e1, f0, f1, dv, w0, w1, w2, bb):
  def body(x_ref, a_ref, d_ref, e0_ref, e1_ref, f0_ref, f1_ref, dv_ref,
           w0_ref, w1_ref, w2_ref, b_ref, or_ref, oi_ref):
    x = x_ref[...]
    a = a_ref[...]
    d = d_ref[...]
    dvv = dv_ref[...]
    e = (e0_ref[...] + e1_ref[...]) * dvv
    f = (f0_ref[...] + f1_ref[...]) * dvv
    dot = lambda u, wref: jnp.dot(u, wref[...],
                                  preferred_element_type=jnp.float32)
    t0 = dot(x, w0_ref)
    outr = t0 + dot(a - d, w1_ref) + dot(2.0 * e - 2.0 * f, w2_ref) + b_ref[...]
    outi = t0 + dot(a + d, w1_ref) + dot(2.0 * e + 2.0 * f - 2.0 * x, w2_ref) \
        + b_ref[...]
    or_ref[...] = jnp.maximum(outr, 0.0)
    oi_ref[...] = outi

  out = jax.ShapeDtypeStruct((NP8, 128), jnp.float32)
  return pl.pallas_call(
      body,
      out_shape=(out, out),
      grid=(pl.cdiv(NP8, BLK),),
      in_specs=[_row_spec()] * 8 + [_full_spec((128, 128))] * 3
      + [_full_spec((1, 128))],
      out_specs=(_row_spec(), _row_spec()))(
          x_p, a_p, d_p, e0, e1, f0, f1, dv, w0, w1, w2, bb)


def _layer2(xr_p, xi_p, a_p, b_p, c_p, d_p, e0, e1, f0, f1, g0, g1, h0, h1,
            dv, w0, w1, w2, bb):
  def body(xr_ref, xi_ref, a_ref, b_ref, c_ref, d_ref, e0_ref, e1_ref,
           f0_ref, f1_ref, g0_ref, g1_ref, h0_ref, h1_ref, dv_ref,
           w0_ref, w1_ref, w2_ref, bias_ref, or_ref, oi_ref):
    xr = xr_ref[...]
    xi = xi_ref[...]
    a = a_ref[...]
    b = b_ref[...]
    c = c_ref[...]
    d = d_ref[...]
    dvv = dv_ref[...]
    e = (e0_ref[...] + e1_ref[...]) * dvv
    f = (f0_ref[...] + f1_ref[...]) * dvv
    g = (g0_ref[...] + g1_ref[...]) * dvv
    h = (h0_ref[...] + h1_ref[...]) * dvv
    dot = lambda u, wref: jnp.dot(u, wref[...],
                                  preferred_element_type=jnp.float32)
    outr = dot(xr, w0_ref) + dot(a - d, w1_ref) \
        + dot(2.0 * e - xr - 2.0 * f + xi, w2_ref) + bias_ref[...]
    outi = dot(xi, w0_ref) + dot(b + c, w1_ref) \
        + dot(2.0 * g - xi + 2.0 * h - xr, w2_ref) + bias_ref[...]
    or_ref[...] = jnp.maximum(outr, 0.0)
    oi_ref[...] = outi

  out = jax.ShapeDtypeStruct((NP8, 128), jnp.float32)
  return pl.pallas_call(
      body,
      out_shape=(out, out),
      grid=(pl.cdiv(NP8, BLK),),
      in_specs=[_row_spec()] * 15 + [_full_spec((128, 128))] * 3
      + [_full_spec((1, 128))],
      out_specs=(_row_spec(), _row_spec()))(
          xr_p, xi_p, a_p, b_p, c_p, d_p, e0, e1, f0, f1, g0, g1, h0, h1,
          dv, w0, w1, w2, bb)


def _layer3_head(xr_p, xi_p, a_p, b_p, c_p, d_p, e0, e1, f0, f1, g0, g1,
                 h0, h1, dv, w0, w1, w2, bb, arh, aih, blb):
  def body(xr_ref, xi_ref, a_ref, b_ref, c_ref, d_ref, e0_ref, e1_ref,
           f0_ref, f1_ref, g0_ref, g1_ref, h0_ref, h1_ref, dv_ref,
           w0_ref, w1_ref, w2_ref, bias_ref, arh_ref, aih_ref, blb_ref,
           o_ref):
    xr = xr_ref[...]
    xi = xi_ref[...]
    a = a_ref[...]
    b = b_ref[...]
    c = c_ref[...]
    d = d_ref[...]
    dvv = dv_ref[...]
    e = (e0_ref[...] + e1_ref[...]) * dvv
    f = (f0_ref[...] + f1_ref[...]) * dvv
    g = (g0_ref[...] + g1_ref[...]) * dvv
    h = (h0_ref[...] + h1_ref[...]) * dvv
    dot = lambda u, wref: jnp.dot(u, wref[...],
                                  preferred_element_type=jnp.float32)
    r3 = dot(xr, w0_ref) + dot(a - d, w1_ref) \
        + dot(2.0 * e - xr - 2.0 * f + xi, w2_ref) + bias_ref[...]
    i3 = dot(xi, w0_ref) + dot(b + c, w1_ref) \
        + dot(2.0 * g - xi + 2.0 * h - xr, w2_ref) + bias_ref[...]
    o_ref[...] = dot(r3, arh_ref) + dot(i3, aih_ref) + blb_ref[...]

  out = jax.ShapeDtypeStruct((NP8, 128), jnp.float32)
  return pl.pallas_call(
      body,
      out_shape=out,
      grid=(pl.cdiv(NP8, BLK),),
      in_specs=[_row_spec()] * 15 + [_full_spec((128, 128))] * 3
      + [_full_spec((1, 128))] + [_full_spec((128, 128))] * 2
      + [_full_spec((1, 128))],
      out_specs=_row_spec())(
          xr_p, xi_p, a_p, b_p, c_p, d_p, e0, e1, f0, f1, g0, g1, h0, h1,
          dv, w0, w1, w2, bb, arh, aih, blb)


def _primes_below(limit, count):
  out = []
  x = limit - 1
  while len(out) < count:
    p = True
    i = 3
    while i * i <= x:
      if x % i == 0:
        p = False
        break
      i += 2
    if (x % 2) and p:
      out.append(x)
    x -= 2
  return out


_TS = 1 << 26
_PRIMES = np.array(_primes_below(_TS, 16), dtype=np.int64)


def _pair_delta(row, col):
  key = row * jnp.int64(N) + col
  rkey = col * jnp.int64(N) + row
  primes = jnp.asarray(_PRIMES)

  def body(state):
    r, delta, resolved = state
    p = primes[jnp.mod(r, 16)]
    s = (key % p).astype(jnp.int32)
    q = (key // p).astype(jnp.int32)
    s2 = (rkey % p).astype(jnp.int32)
    q2 = (rkey // p).astype(jnp.int32)
    cnt = jnp.zeros((_TS,), jnp.int32).at[s].add(1)
    qmin = jnp.full((_TS,), 2**31 - 1, jnp.int32).at[s].min(q)
    qmax = jnp.full((_TS,), -1, jnp.int32).at[s].max(q)
    own_clean = qmin[s] == qmax[s]
    rev_cnt = cnt[s2]
    rev_clean = (qmin[s2] == qmax[s2]) | (rev_cnt == 0)
    c_ab = cnt[s]
    c_ba = jnp.where((rev_cnt > 0) & (qmin[s2] == q2), rev_cnt, 0)
    ok = own_clean & rev_clean & (~resolved)
    delta = jnp.where(ok, c_ab - c_ba, delta)
    resolved = resolved | ok | (r >= 31)
    return (r + jnp.int32(1), delta, resolved)

  def cond(state):
    return ~jnp.all(state[2])

  e_cnt = row.shape[0]
  init = (jnp.int32(0), jnp.ones((e_cnt,), jnp.int32),
          jnp.zeros((e_cnt,), bool))
  _, delta, _ = lax.while_loop(cond, body, init)
  return delta


def _build_edges(edge_index):
  e_cnt = edge_index.shape[1]
  m = 2 * e_cnt
  row = edge_index[0]
  col = edge_index[1]
  r_all = jnp.concatenate([row, col]).astype(jnp.int32)
  c_all = jnp.concatenate([col, row]).astype(jnp.int32)
  delta = _pair_delta(row, col)
  d_e = jnp.concatenate([delta, -delta])
  dm4 = jnp.mod(d_e, 4)

  deg = jax.ops.segment_sum(jnp.full((m,), 0.5, jnp.float32), r_all,
                            num_segments=N)
  dinv = jnp.where(deg > 0, 1.0 / jnp.sqrt(jnp.maximum(deg, 1e-30)),
                   0.0).astype(jnp.float32)

  i_all = jnp.arange(m, dtype=jnp.int32)
  zrow = ZB + (i_all & (NZ - 1))
  src_r = jnp.where(dm4 == 0, r_all, jnp.where(dm4 == 2, r_all + SEC, zrow))
  src_i = jnp.where(dm4 == 1, r_all, jnp.where(dm4 == 3, r_all + SEC, zrow))
  dst = c_all

  nmac = -(-m // (NW * MW * KCH))
  m_pad = NW * MW * KCH * nmac
  pad_n = m_pad - m
  pidx = jnp.arange(pad_n, dtype=jnp.int32)
  zpad = ZB + (pidx & (NZ - 1))
  src_r = jnp.concatenate([src_r, zpad])
  src_i = jnp.concatenate([src_i, zpad])
  dst = jnp.concatenate([dst, N + (pidx % 96)])
  return (src_r.reshape(-1, MW), src_i.reshape(-1, MW),
          dst.reshape(-1, MW), dinv, nmac)


def _blockdiag(w):
  wp = jnp.pad(w, ((0, 16 - w.shape[0]), (0, 16 - w.shape[1])))
  return jnp.kron(jnp.eye(8, dtype=jnp.float32), wp)


def kernel(data_x, data_edge_index, W1, b1, W2, b2, W3, b3, Wl, bl):
  x = data_x.astype(jnp.float32)
  srcr2, srci2, dst2, dinv, nmac = _build_edges(data_edge_index)
  zeros = jnp.zeros((NACC, F), jnp.float32)
  dv = jnp.repeat(dinv, F).reshape(NP8, 128)

  w1b = [_blockdiag(W1[k]) for k in range(3)]
  w2b = [_blockdiag(W2[k]) for k in range(3)]
  w3b = [_blockdiag(W3[k]) for k in range(3)]
  b1b = jnp.tile(jnp.pad(b1, (0, 16 - b1.shape[0])), 8)[None, :]
  b2b = jnp.tile(jnp.pad(b2, (0, 16 - b2.shape[0])), 8)[None, :]
  b3b = jnp.tile(jnp.pad(b3, (0, 16 - b3.shape[0])), 8)[None, :]
  bh_r = jnp.zeros((16, 16), jnp.float32).at[:8, 0].set(Wl[:8, 0])
  bh_i = jnp.zeros((16, 16), jnp.float32).at[:8, 0].set(Wl[8:, 0])
  arh = jnp.kron(jnp.eye(8, dtype=jnp.float32), bh_r)
  aih = jnp.kron(jnp.eye(8, dtype=jnp.float32), bh_i)
  blb = jnp.tile(jnp.concatenate([bl, jnp.zeros((15,), jnp.float32)]),
                 8)[None, :]

  sc_pass = _make_sc_pass(nmac)
  src_stack = jnp.stack([srcr2, srci2])

  def scan_passes(tbls_p, sel):
    tbl_stack = tbls_p.reshape(tbls_p.shape[0], TH, F)

    def step(carry, s):
      src2 = lax.dynamic_index_in_dim(src_stack, s[0], 0, keepdims=False)
      tbl = lax.dynamic_index_in_dim(tbl_stack, s[1], 0, keepdims=False)
      parts = sc_pass(src2, dst2, tbl, zeros)
      return carry, parts

    _, outs = lax.scan(step, jnp.int32(0), sel)
    return [(outs[l, 0, :N].reshape(NP8, 128),
             outs[l, 1, :N].reshape(NP8, 128))
            for l in range(sel.shape[0])]

  x_p = x.reshape(NP8, 128)
  tbl_x = _stack1(x_p, dv)
  (a0, a1), (d0, d1) = scan_passes(
      jnp.stack([tbl_x]), jnp.array([[0, 0], [1, 0]], jnp.int32))
  tblA, a_p = _stackc(a0, a1, dv)
  tblD, d_p = _stackc(d0, d1, dv)
  (e0, e1), (f0, f1) = scan_passes(
      jnp.stack([tblA, tblD]), jnp.array([[0, 0], [1, 1]], jnp.int32))
  xr_p, xi_p = _layer1(x_p, a_p, d_p, e0, e1, f0, f1, dv,
                       w1b[0], w1b[1], w1b[2], b1b)

  head_p = None
  for wlist, bb, last in ((w2b, b2b, False), (w3b, b3b, True)):
    tbl_r = _stack1(xr_p, dv)
    tbl_i = _stack1(xi_p, dv)
    (a0, a1), (b0, b1_), (c0, c1), (d0, d1) = scan_passes(
        jnp.stack([tbl_r, tbl_i]),
        jnp.array([[0, 0], [0, 1], [1, 0], [1, 1]], jnp.int32))
    tblA, a_p = _stackc(a0, a1, dv)
    tblB, b_p = _stackc(b0, b1_, dv)
    tblC, c_p = _stackc(c0, c1, dv)
    tblD, d_p = _stackc(d0, d1, dv)
    (e0, e1), (g0, g1), (h0, h1), (f0, f1) = scan_passes(
        jnp.stack([tblA, tblB, tblC, tblD]),
        jnp.array([[0, 0], [0, 1], [1, 2], [1, 3]], jnp.int32))
    if last:
      head_p = _layer3_head(xr_p, xi_p, a_p, b_p, c_p, d_p,
                            e0, e1, f0, f1, g0, g1, h0, h1, dv,
                            wlist[0], wlist[1], wlist[2], bb, arh, aih, blb)
    else:
      xr_p, xi_p = _layer2(xr_p, xi_p, a_p, b_p, c_p, d_p,
                           e0, e1, f0, f1, g0, g1, h0, h1, dv,
                           wlist[0], wlist[1], wlist[2], bb)

  return head_p.reshape(N, F)[:, :1]
